# baseline (speedup 1.0000x reference)
import dataclasses
import functools

import jax
import jax.numpy as jnp
from jax import lax
from jax.experimental import pallas as pl
from jax.experimental.pallas import tpu as pltpu
from jax.experimental.pallas import tpu_sc as plsc

_L = 16
_CH = 128
_D = 128


def _norm_body(x_ref, o_ref):
    x = x_ref[...]
    ss = jnp.sum(x * x, axis=1, keepdims=True)
    n = jnp.sqrt(ss)
    o_ref[...] = x / jnp.maximum(n, 1e-12)


def _normalize_rows(x):
    return pl.pallas_call(
        _norm_body,
        out_shape=jax.ShapeDtypeStruct(x.shape, x.dtype),
    )(x)


def _bce_body(n_valid, p_ref, t_ref, o_ref):
    p = p_ref[...]
    t = t_ref[...]
    term = jnp.maximum(p, 0.0) - p * t + jnp.log1p(jnp.exp(-jnp.abs(p)))
    rows, cols = p.shape
    idx = (lax.broadcasted_iota(jnp.int32, (rows, cols), 0) * cols
           + lax.broadcasted_iota(jnp.int32, (rows, cols), 1))
    term = jnp.where(idx < n_valid, term, 0.0)
    o_ref[...] = jnp.sum(term, axis=0, keepdims=True)


def _bce_sum(p2d, t2d, n_valid):
    part = pl.pallas_call(
        functools.partial(_bce_body, n_valid),
        out_shape=jax.ShapeDtypeStruct((1, p2d.shape[1]), jnp.float32),
    )(p2d, t2d)
    return jnp.sum(part)


def _make_edge_kernel(nw, ppi_chunks, dti_chunks):
    ppw = ppi_chunks * _CH
    dtw = dti_chunks * _CH
    mesh = plsc.VectorSubcoreMesh(core_axis_name="c", subcore_axis_name="s")
    info = plsc.get_sparse_core_info()
    nc = info.num_cores

    cp = pltpu.CompilerParams()
    if "needs_layout_passes" in pltpu.CompilerParams.__dataclass_fields__:
        cp = dataclasses.replace(cp, needs_layout_passes=False)

    @functools.partial(
        pl.kernel,
        mesh=mesh,
        compiler_params=cp,
        out_type=jax.ShapeDtypeStruct((nw, 2, _L), jnp.float32),
        scratch_types=[
            pltpu.VMEM((ppw,), jnp.int32),
            pltpu.VMEM((ppw,), jnp.int32),
            pltpu.VMEM((dtw,), jnp.int32),
            pltpu.VMEM((dtw,), jnp.int32),
            pltpu.VMEM((2, _CH, _D), jnp.float32),
            pltpu.VMEM((2, _CH, _D), jnp.float32),
            pltpu.VMEM((2, _L), jnp.float32),
            pltpu.SemaphoreType.DMA,
            pltpu.SemaphoreType.DMA,
            pltpu.SemaphoreType.DMA,
            pltpu.SemaphoreType.DMA,
        ],
    )
    def edge_kernel(gene_hbm, drug_hbm, ps_hbm, pd_hbm, ds_hbm, dd_hbm,
                    out_hbm, psidx, pdidx, dsidx, ddidx, srows, drows, ovec,
                    ss0, ss1, sd0, sd1):
        wid = lax.axis_index("s") * nc + lax.axis_index("c")
        ssems = (ss0, ss1)
        dsems = (sd0, sd1)

        pltpu.sync_copy(ps_hbm.at[pl.ds(wid * ppw, ppw)], psidx)
        pltpu.sync_copy(pd_hbm.at[pl.ds(wid * ppw, ppw)], pdidx)
        pltpu.sync_copy(ds_hbm.at[pl.ds(wid * dtw, dtw)], dsidx)
        pltpu.sync_copy(dd_hbm.at[pl.ds(wid * dtw, dtw)], ddidx)

        def start(tbl_s, tbl_d, sidx, didx, c, b):
            pltpu.async_copy(tbl_s.at[sidx.at[pl.ds(c * _CH, _CH)]],
                             srows.at[b], ssems[b])
            pltpu.async_copy(tbl_d.at[didx.at[pl.ds(c * _CH, _CH)]],
                             drows.at[b], dsems[b])

        def wait(tbl_s, tbl_d, sidx, didx, b):
            pltpu.make_async_copy(tbl_s.at[sidx.at[pl.ds(0, _CH)]],
                                  srows.at[b], ssems[b]).wait()
            pltpu.make_async_copy(tbl_d.at[didx.at[pl.ds(0, _CH)]],
                                  drows.at[b], dsems[b]).wait()

        def compute(b, acc):
            sb = srows.at[b]
            db = drows.at[b]

            def group(g, acc):
                row = g * _L + lax.iota(jnp.int32, _L)
                dots = jnp.zeros((_L,), jnp.float32)
                for k in range(_D):
                    col = jnp.full((_L,), k, jnp.int32)
                    s = plsc.load_gather(sb, [row, col])
                    t = plsc.load_gather(db, [row, col])
                    dots = dots + s * t
                r = dots - 1.0
                return acc + r * r

            return lax.fori_loop(0, _CH // _L, group, acc)

        def run_class(tbl_s, tbl_d, sidx, didx, nch, acc):
            start(tbl_s, tbl_d, sidx, didx, 0, 0)
            start(tbl_s, tbl_d, sidx, didx, 1, 1)

            def pair(i, acc):
                for b in (0, 1):
                    c = i * 2 + b
                    wait(tbl_s, tbl_d, sidx, didx, b)
                    acc = compute(b, acc)

                    @pl.when(c + 2 < nch)
                    def _():
                        start(tbl_s, tbl_d, sidx, didx, c + 2, b)
                return acc

            return lax.fori_loop(0, nch // 2, pair, acc)

        acc_ppi = run_class(gene_hbm, gene_hbm, psidx, pdidx, ppi_chunks,
                            jnp.zeros((_L,), jnp.float32))
        acc_dti = run_class(drug_hbm, gene_hbm, dsidx, ddidx, dti_chunks,
                            jnp.zeros((_L,), jnp.float32))

        ovec[0, :] = acc_ppi
        ovec[1, :] = acc_dti
        pltpu.sync_copy(ovec, out_hbm.at[wid])

    return edge_kernel


def _pad_idx(idx, total, fill):
    pad = total - idx.shape[0]
    if pad == 0:
        return idx.astype(jnp.int32)
    return jnp.concatenate(
        [idx.astype(jnp.int32),
         jnp.full((pad,), fill, dtype=jnp.int32)])


def kernel(gene_x, drug_x, predicted_dti, known_dti, ppi_edge_index,
           dti_src, dti_dst):
    dti_weight = 1.0
    topology_weight = 0.1

    n_gene, d = gene_x.shape
    n_drug = drug_x.shape[0]
    e_ppi = ppi_edge_index.shape[1]
    e_dti = predicted_dti.shape[0]

    info = plsc.get_sparse_core_info()
    nw = info.num_cores * info.num_subcores

    drug_rows = ((n_drug + _CH - 1) // _CH) * _CH + _CH
    drug_pad = jnp.concatenate(
        [drug_x, jnp.zeros((drug_rows - n_drug, d), drug_x.dtype)])
    gene_n = _normalize_rows(gene_x)
    drug_n = _normalize_rows(drug_pad)

    cols = 128
    n_flat = ((e_dti + cols * 8 - 1) // (cols * 8)) * (cols * 8)
    p2d = jnp.pad(predicted_dti, (0, n_flat - e_dti)).reshape(-1, cols)
    t2d = jnp.pad(known_dti, (0, n_flat - e_dti)).reshape(-1, cols)
    bce_total = _bce_sum(p2d, t2d, e_dti)

    per_block = nw * _CH

    def _even_chunks(n):
        c = (n + per_block - 1) // per_block
        return c + (c % 2)

    ppi_chunks = _even_chunks(e_ppi)
    dti_chunks = _even_chunks(e_dti)
    ppi_total = ppi_chunks * per_block
    dti_total = dti_chunks * per_block
    dti_pad = dti_total - e_dti

    ps = _pad_idx(ppi_edge_index[0], ppi_total, 0)
    pd = _pad_idx(ppi_edge_index[1], ppi_total, 0)
    ds = _pad_idx(dti_src, dti_total, n_drug)
    dd = _pad_idx(dti_dst, dti_total, 0)

    edge_kernel = _make_edge_kernel(nw, ppi_chunks, dti_chunks)
    parts = edge_kernel(gene_n, drug_n, ps, pd, ds, dd)

    ppi_sum = jnp.sum(parts[:, 0, :])
    dti_sum = jnp.sum(parts[:, 1, :]) - jnp.float32(dti_pad)

    topology_loss = ppi_sum / e_ppi + dti_sum / e_dti
    dti_loss = bce_total / e_dti
    return dti_weight * dti_loss + topology_weight * topology_loss


# speedup vs baseline: 1.3919x; 1.3919x over previous
import dataclasses
import functools

import jax
import jax.numpy as jnp
from jax import lax
from jax.experimental import pallas as pl
from jax.experimental.pallas import tpu as pltpu
from jax.experimental.pallas import tpu_sc as plsc

_L = 16
_CH = 128
_D = 128


def _norm_body(x_ref, o_ref):
    x = x_ref[...]
    ss = jnp.sum(x * x, axis=1, keepdims=True)
    n = jnp.sqrt(ss)
    o_ref[...] = x / jnp.maximum(n, 1e-12)


def _normalize_rows(x):
    return pl.pallas_call(
        _norm_body,
        out_shape=jax.ShapeDtypeStruct(x.shape, x.dtype),
    )(x)


def _bce_body(n_valid, p_ref, t_ref, o_ref):
    p = p_ref[...]
    t = t_ref[...]
    term = jnp.maximum(p, 0.0) - p * t + jnp.log1p(jnp.exp(-jnp.abs(p)))
    rows, cols = p.shape
    idx = (lax.broadcasted_iota(jnp.int32, (rows, cols), 0) * cols
           + lax.broadcasted_iota(jnp.int32, (rows, cols), 1))
    term = jnp.where(idx < n_valid, term, 0.0)
    o_ref[...] = jnp.sum(term, axis=0, keepdims=True)


def _bce_sum(p2d, t2d, n_valid):
    part = pl.pallas_call(
        functools.partial(_bce_body, n_valid),
        out_shape=jax.ShapeDtypeStruct((1, p2d.shape[1]), jnp.float32),
    )(p2d, t2d)
    return jnp.sum(part)


def _make_edge_kernel(nw, ppi_chunks, dti_chunks):
    ppw = ppi_chunks * _CH
    dtw = dti_chunks * _CH
    mesh = plsc.VectorSubcoreMesh(core_axis_name="c", subcore_axis_name="s")
    info = plsc.get_sparse_core_info()
    nc = info.num_cores

    cp = pltpu.CompilerParams()
    if "needs_layout_passes" in pltpu.CompilerParams.__dataclass_fields__:
        cp = dataclasses.replace(cp, needs_layout_passes=False)

    @functools.partial(
        pl.kernel,
        mesh=mesh,
        compiler_params=cp,
        out_type=jax.ShapeDtypeStruct((nw, 2, _L), jnp.float32),
        scratch_types=[
            pltpu.VMEM((ppw,), jnp.int32),
            pltpu.VMEM((ppw,), jnp.int32),
            pltpu.VMEM((dtw,), jnp.int32),
            pltpu.VMEM((dtw,), jnp.int32),
            pltpu.VMEM((2, _CH, _D), jnp.float32),
            pltpu.VMEM((2, _CH, _D), jnp.float32),
            pltpu.VMEM((2, _L), jnp.float32),
            pltpu.SemaphoreType.DMA,
            pltpu.SemaphoreType.DMA,
            pltpu.SemaphoreType.DMA,
            pltpu.SemaphoreType.DMA,
        ],
    )
    def edge_kernel(gene_hbm, drug_hbm, ps_hbm, pd_hbm, ds_hbm, dd_hbm,
                    out_hbm, psidx, pdidx, dsidx, ddidx, srows, drows, ovec,
                    ss0, ss1, sd0, sd1):
        wid = lax.axis_index("s") * nc + lax.axis_index("c")
        ssems = (ss0, ss1)
        dsems = (sd0, sd1)

        pltpu.sync_copy(ps_hbm.at[pl.ds(wid * ppw, ppw)], psidx)
        pltpu.sync_copy(pd_hbm.at[pl.ds(wid * ppw, ppw)], pdidx)
        pltpu.sync_copy(ds_hbm.at[pl.ds(wid * dtw, dtw)], dsidx)
        pltpu.sync_copy(dd_hbm.at[pl.ds(wid * dtw, dtw)], ddidx)

        def start(tbl_s, tbl_d, sidx, didx, c, b):
            pltpu.async_copy(tbl_s.at[sidx.at[pl.ds(c * _CH, _CH)]],
                             srows.at[b], ssems[b])
            pltpu.async_copy(tbl_d.at[didx.at[pl.ds(c * _CH, _CH)]],
                             drows.at[b], dsems[b])

        def wait(tbl_s, tbl_d, sidx, didx, b):
            pltpu.make_async_copy(tbl_s.at[sidx.at[pl.ds(0, _CH)]],
                                  srows.at[b], ssems[b]).wait()
            pltpu.make_async_copy(tbl_d.at[didx.at[pl.ds(0, _CH)]],
                                  drows.at[b], dsems[b]).wait()

        def compute(b, acc):
            sb = srows.at[b]
            db = drows.at[b]

            def edge(e, acc):
                prod = sb[e, pl.ds(0, _L)] * db[e, pl.ds(0, _L)]
                for k in range(1, _D // _L):
                    prod = prod + (sb[e, pl.ds(k * _L, _L)]
                                   * db[e, pl.ds(k * _L, _L)])
                dt = jnp.sum(prod)
                r = dt - 1.0
                return acc + r * r

            return lax.fori_loop(0, _CH, edge, acc)

        def run_class(tbl_s, tbl_d, sidx, didx, nch, acc):
            start(tbl_s, tbl_d, sidx, didx, 0, 0)
            start(tbl_s, tbl_d, sidx, didx, 1, 1)

            def pair(i, acc):
                for b in (0, 1):
                    c = i * 2 + b
                    wait(tbl_s, tbl_d, sidx, didx, b)
                    acc = compute(b, acc)

                    @pl.when(c + 2 < nch)
                    def _():
                        start(tbl_s, tbl_d, sidx, didx, c + 2, b)
                return acc

            return lax.fori_loop(0, nch // 2, pair, acc)

        acc_ppi = run_class(gene_hbm, gene_hbm, psidx, pdidx, ppi_chunks,
                            jnp.zeros((), jnp.float32))
        acc_dti = run_class(drug_hbm, gene_hbm, dsidx, ddidx, dti_chunks,
                            jnp.zeros((), jnp.float32))

        lane = lax.iota(jnp.int32, _L)
        ovec[0, :] = jnp.where(lane == 0, acc_ppi, 0.0)
        ovec[1, :] = jnp.where(lane == 0, acc_dti, 0.0)
        pltpu.sync_copy(ovec, out_hbm.at[wid])

    return edge_kernel


def _pad_idx(idx, total, fill):
    pad = total - idx.shape[0]
    if pad == 0:
        return idx.astype(jnp.int32)
    return jnp.concatenate(
        [idx.astype(jnp.int32),
         jnp.full((pad,), fill, dtype=jnp.int32)])


def kernel(gene_x, drug_x, predicted_dti, known_dti, ppi_edge_index,
           dti_src, dti_dst):
    dti_weight = 1.0
    topology_weight = 0.1

    n_gene, d = gene_x.shape
    n_drug = drug_x.shape[0]
    e_ppi = ppi_edge_index.shape[1]
    e_dti = predicted_dti.shape[0]

    info = plsc.get_sparse_core_info()
    nw = info.num_cores * info.num_subcores

    drug_rows = ((n_drug + _CH - 1) // _CH) * _CH + _CH
    drug_pad = jnp.concatenate(
        [drug_x, jnp.zeros((drug_rows - n_drug, d), drug_x.dtype)])
    gene_n = _normalize_rows(gene_x)
    drug_n = _normalize_rows(drug_pad)

    cols = 128
    n_flat = ((e_dti + cols * 8 - 1) // (cols * 8)) * (cols * 8)
    p2d = jnp.pad(predicted_dti, (0, n_flat - e_dti)).reshape(-1, cols)
    t2d = jnp.pad(known_dti, (0, n_flat - e_dti)).reshape(-1, cols)
    bce_total = _bce_sum(p2d, t2d, e_dti)

    per_block = nw * _CH

    def _even_chunks(n):
        c = (n + per_block - 1) // per_block
        return c + (c % 2)

    ppi_chunks = _even_chunks(e_ppi)
    dti_chunks = _even_chunks(e_dti)
    ppi_total = ppi_chunks * per_block
    dti_total = dti_chunks * per_block
    dti_pad = dti_total - e_dti

    ps = _pad_idx(ppi_edge_index[0], ppi_total, 0)
    pd = _pad_idx(ppi_edge_index[1], ppi_total, 0)
    ds = _pad_idx(dti_src, dti_total, n_drug)
    dd = _pad_idx(dti_dst, dti_total, 0)

    edge_kernel = _make_edge_kernel(nw, ppi_chunks, dti_chunks)
    parts = edge_kernel(gene_n, drug_n, ps, pd, ds, dd)

    ppi_sum = jnp.sum(parts[:, 0, :])
    dti_sum = jnp.sum(parts[:, 1, :]) - jnp.float32(dti_pad)

    topology_loss = ppi_sum / e_ppi + dti_sum / e_dti
    dti_loss = bce_total / e_dti
    return dti_weight * dti_loss + topology_weight * topology_loss
